# baseline (speedup 1.0000x reference)
import functools
import math

import jax
import jax.numpy as jnp
from jax import lax
from jax.experimental import pallas as pl
from jax.experimental.pallas import tpu as pltpu
import jax.experimental.pallas.tpu_sc as plsc

_TAU = 4
_K = 8
_ROWS = 256


def _pack_bf16(e, o):
    eb = lax.bitcast_convert_type(e.astype(jnp.bfloat16),
                                  jnp.uint16).astype(jnp.uint32)
    ob = lax.bitcast_convert_type(o.astype(jnp.bfloat16),
                                  jnp.uint16).astype(jnp.uint32)
    return lax.bitcast_convert_type((ob << 16) | eb, jnp.int32)


def _qkv_body(h_last_ref, hist_ref, wq_ref, bq_ref, wk_ref, bk_ref,
              wv_ref, bv_ref, qcat_ref, kv_ref, *, tau, d):
    dw = d // 2
    n = h_last_ref.shape[2]
    fio = lax.broadcasted_iota(jnp.int32, (d, dw), 0)
    wio = lax.broadcasted_iota(jnp.int32, (d, dw), 1)
    sel_e = (fio == 2 * wio).astype(jnp.float32)
    sel_o = (fio == 2 * wio + 1).astype(jnp.float32)

    def proj(x, w_ref, b_ref, sel):
        w = jnp.dot(w_ref[...], sel, preferred_element_type=jnp.float32)
        bias = jnp.dot(b_ref[...], sel, preferred_element_type=jnp.float32)
        return jnp.dot(x, w, preferred_element_type=jnp.float32) + bias

    h_last = h_last_ref[0, 0]
    hist = hist_ref[0].reshape(tau * n, d)
    qe = proj(h_last, wq_ref, bq_ref, sel_e)
    qo = proj(h_last, wq_ref, bq_ref, sel_o)
    qcat_ref[0] = jnp.concatenate([qe] * tau + [qo] * tau, axis=-1)
    kwords = _pack_bf16(proj(hist, wk_ref, bk_ref, sel_e),
                        proj(hist, wk_ref, bk_ref, sel_o))
    vwords = _pack_bf16(proj(hist, wv_ref, bv_ref, sel_e),
                        proj(hist, wv_ref, bv_ref, sel_o))
    pieces = ([kwords[t * n:(t + 1) * n] for t in range(tau)]
              + [vwords[t * n:(t + 1) * n] for t in range(tau)])
    kv_ref[0] = jnp.concatenate(pieces, axis=-1)


def _topk_body(windc_ref, windr_ref, fidx_ref, *, n, tau, k):
    b = pl.program_id(0)
    wcx = windc_ref[0, 0, :, 4:5]
    wcy = windc_ref[0, 0, :, 5:6]
    wrx = windr_ref[0, 0:1, :]
    wry = windr_ref[0, 1:2, :]
    dx = wcx - wrx
    dy = wcy - wry
    d2 = dx * dx + dy * dy
    rows = d2.shape[0]
    iota = lax.broadcasted_iota(jnp.int32, (rows, n), 1)
    cols = []
    for _ in range(k):
        j = jnp.argmin(d2, axis=1, keepdims=True).astype(jnp.int32)
        d2 = jnp.where(iota == j, jnp.inf, d2)
        cols.append(j + b * n)
    fidx_ref[0] = jnp.concatenate(cols, axis=1)


def _sc_gather_body(kv_hbm, idx_hbm, out_hbm, idx_v, rows_v, sem, *,
                    rows_per_worker, chunk, num_cores):
    wid = lax.axis_index("s") * num_cores + lax.axis_index("c")
    nchunks = rows_per_worker // chunk

    def body(c, carry):
        base = wid * rows_per_worker + c * chunk
        pltpu.sync_copy(idx_hbm.at[pl.ds(base, chunk)], idx_v)
        pltpu.async_copy(kv_hbm.at[idx_v], rows_v, sem).wait()
        pltpu.sync_copy(rows_v, out_hbm.at[pl.ds(base, chunk)])
        return carry

    lax.fori_loop(0, nchunks, body, 0)


def _attn_body(qcat_ref, g_ref, w1_ref, b1_ref, w2_ref, b2_ref,
               out_ref, *, d, tau, k):
    dw = d // 2
    td = tau * dw
    qcat = qcat_ref[0]
    g = g_ref[...]
    rows = qcat.shape[0]
    rk = rows * k
    hi_mask = jnp.int32(-65536)

    def lo(w):
        return lax.bitcast_convert_type(lax.shift_left(w, 16), jnp.float32)

    def hi(w):
        return lax.bitcast_convert_type(w & hi_mask, jnp.float32)

    def rep_rows(x):
        return jnp.broadcast_to(x[:, None, :],
                                (rows, k, x.shape[-1])).reshape(rk, x.shape[-1])

    lane = lax.broadcasted_iota(jnp.int32, (td, tau), 0)
    tcol = lax.broadcasted_iota(jnp.int32, (td, tau), 1)
    seg = (lane // dw == tcol).astype(jnp.float32)
    lane2 = lax.broadcasted_iota(jnp.int32, (tau, td), 1)
    trow = lax.broadcasted_iota(jnp.int32, (tau, td), 0)
    expand = (lane2 // dw == trow).astype(jnp.float32)
    lane3 = lax.broadcasted_iota(jnp.int32, (td, dw), 0)
    fcol = lax.broadcasted_iota(jnp.int32, (td, dw), 1)
    fold = (lane3 % dw == fcol).astype(jnp.float32)

    qet = rep_rows(qcat[:, :td])
    qot = rep_rows(qcat[:, td:])
    kw = g[:, :td]
    vw = g[:, td:]
    comb = qet * lo(kw) + qot * hi(kw)
    s4 = jnp.dot(comb, seg,
                 preferred_element_type=jnp.float32) / math.sqrt(d)
    s4g = s4.reshape(rows, k, tau)
    m = jnp.max(jnp.max(s4g, axis=2, keepdims=True), axis=1, keepdims=True)
    e = jnp.exp(s4g - m)
    ssum = jnp.sum(jnp.sum(e, axis=2, keepdims=True), axis=1, keepdims=True)
    w4 = (e / ssum).reshape(rk, tau)
    w128 = jnp.dot(w4, expand, preferred_element_type=jnp.float32)
    acc_e = jnp.dot(w128 * lo(vw), fold,
                    preferred_element_type=jnp.float32)
    acc_o = jnp.dot(w128 * hi(vw), fold,
                    preferred_element_type=jnp.float32)
    ctx_e = jnp.sum(acc_e.reshape(rows, k, dw), axis=1)
    ctx_o = jnp.sum(acc_o.reshape(rows, k, dw), axis=1)
    ctx = jnp.concatenate([ctx_e, ctx_o], axis=-1)
    hid = jnp.dot(ctx, w1_ref[...],
                  preferred_element_type=jnp.float32) + b1_ref[...]
    hid = 0.5 * hid * (1.0 + lax.erf(hid / math.sqrt(2.0)))
    out_ref[...] = jnp.dot(hid, w2_ref[...],
                           preferred_element_type=jnp.float32) + b2_ref[...]


def kernel(h_e, x_orig, Wq, bq, Wk, bk, Wv, bv, W1, b1, W2, b2):
    b, T, n, d = h_e.shape
    t0 = T - 1
    t_start = max(0, t0 - _TAU + 1)
    tau = t0 - t_start + 1
    k = min(_K, n)
    rows = _ROWS
    nb = n // rows
    kt = k * tau

    F = x_orig.shape[-1]
    windr = jnp.transpose(x_orig[t0, :, :, 4:6], (0, 2, 1))

    full = lambda shp: pl.BlockSpec(shp, lambda *_: (0,) * len(shp))

    na = 4
    nblk = n // na
    qcat, kv_packed = pl.pallas_call(
        functools.partial(_qkv_body, tau=tau, d=d),
        grid=(b, na),
        in_specs=[
            pl.BlockSpec((1, 1, nblk, d), lambda i, r: (i, t0, r, 0)),
            pl.BlockSpec((1, tau, nblk, d),
                         lambda i, r: (i, t_start // tau, r, 0)),
            full((d, d)), full((1, d)), full((d, d)), full((1, d)),
            full((d, d)), full((1, d)),
        ],
        out_specs=[
            pl.BlockSpec((1, nblk, tau * d), lambda i, r: (i, r, 0)),
            pl.BlockSpec((1, nblk, tau * d), lambda i, r: (i, r, 0)),
        ],
        out_shape=[
            jax.ShapeDtypeStruct((b, n, tau * d), jnp.float32),
            jax.ShapeDtypeStruct((b, n, tau * d), jnp.int32),
        ],
    )(h_e, h_e, Wq, bq.reshape(1, d),
      Wk, bk.reshape(1, d), Wv, bv.reshape(1, d))
    kv_packed = kv_packed.reshape(b * n, tau * d)

    fidx = pl.pallas_call(
        functools.partial(_topk_body, n=n, tau=tau, k=k),
        grid=(b, nb),
        in_specs=[
            pl.BlockSpec((1, 1, rows, F), lambda i, r: (t0, i, r, 0)),
            pl.BlockSpec((1, 2, n), lambda i, r: (i, 0, 0)),
        ],
        out_specs=pl.BlockSpec((1, rows, k), lambda i, r: (i, r, 0)),
        out_shape=jax.ShapeDtypeStruct((b, n, k), jnp.int32),
    )(x_orig, windr)

    try:
        info = plsc.get_sparse_core_info()
        num_cores, num_subcores = info.num_cores, info.num_subcores
    except ValueError:
        num_cores, num_subcores = 2, 16
    nw = num_cores * num_subcores
    batch_idx = n * k
    rpw = batch_idx // nw
    chunk = 128
    mesh = plsc.VectorSubcoreMesh(core_axis_name="c", subcore_axis_name="s",
                                  num_cores=num_cores,
                                  num_subcores=num_subcores)
    sc_gather = pl.kernel(
        functools.partial(_sc_gather_body, rows_per_worker=rpw, chunk=chunk,
                          num_cores=num_cores),
        out_type=jax.ShapeDtypeStruct((batch_idx, tau * d), jnp.int32),
        mesh=mesh,
        scratch_types=[
            pltpu.VMEM((chunk,), jnp.int32),
            pltpu.VMEM((chunk, tau * d), jnp.int32),
            pltpu.SemaphoreType.DMA,
        ],
    )

    w1p = jnp.concatenate([W1[0::2], W1[1::2]], axis=0)

    def attn_call(bi, g_b):
        return pl.pallas_call(
            functools.partial(_attn_body, d=d, tau=tau, k=k),
            grid=(nb,),
            in_specs=[
                pl.BlockSpec((1, rows, tau * d), lambda r: (bi, r, 0)),
                pl.BlockSpec((rows * k, tau * d), lambda r: (r, 0)),
                full((d, d)), full((d,)), full((d, d)), full((d,)),
            ],
            out_specs=pl.BlockSpec((rows, d), lambda r: (r, 0)),
            out_shape=jax.ShapeDtypeStruct((n, d), jnp.float32),
        )(qcat, g_b, w1p, b1, W2, b2)

    fidx_flat = fidx.reshape(b, batch_idx)
    outs = []
    for bi in range(b):
        g_b = sc_gather(kv_packed, fidx_flat[bi])
        outs.append(attn_call(bi, g_b))
    return jnp.stack(outs, axis=0)


# speedup vs baseline: 1.1813x; 1.1813x over previous
import functools
import math

import jax
import jax.numpy as jnp
from jax import lax
from jax.experimental import pallas as pl
from jax.experimental.pallas import tpu as pltpu
import jax.experimental.pallas.tpu_sc as plsc

_TAU = 4
_K = 8
_ROWS = 256


def _pack_bf16(e, o):
    eb = lax.bitcast_convert_type(e.astype(jnp.bfloat16),
                                  jnp.uint16).astype(jnp.uint32)
    ob = lax.bitcast_convert_type(o.astype(jnp.bfloat16),
                                  jnp.uint16).astype(jnp.uint32)
    return lax.bitcast_convert_type((ob << 16) | eb, jnp.int32)


def _qkv_body(h_last_ref, hist_ref, wq_ref, bq_ref, wk_ref, bk_ref,
              wv_ref, bv_ref, qcat_ref, kv_ref, *, tau, d):
    dw = d // 2
    n = h_last_ref.shape[1]
    fio = lax.broadcasted_iota(jnp.int32, (d, dw), 0)
    wio = lax.broadcasted_iota(jnp.int32, (d, dw), 1)
    sel_e = (fio == 2 * wio).astype(jnp.float32)
    sel_o = (fio == 2 * wio + 1).astype(jnp.float32)

    def proj(x, w_ref, b_ref, sel):
        w = jnp.dot(w_ref[...], sel, preferred_element_type=jnp.float32)
        bias = jnp.dot(b_ref[...], sel, preferred_element_type=jnp.float32)
        return jnp.dot(x, w, preferred_element_type=jnp.float32) + bias

    h_last = h_last_ref[0]
    hist = hist_ref[0].reshape(tau * n, d)
    qe = proj(h_last, wq_ref, bq_ref, sel_e)
    qo = proj(h_last, wq_ref, bq_ref, sel_o)
    qcat_ref[0] = jnp.concatenate([qe] * tau + [qo] * tau, axis=-1)
    kwords = _pack_bf16(proj(hist, wk_ref, bk_ref, sel_e),
                        proj(hist, wk_ref, bk_ref, sel_o))
    vwords = _pack_bf16(proj(hist, wv_ref, bv_ref, sel_e),
                        proj(hist, wv_ref, bv_ref, sel_o))
    pieces = ([kwords[t * n:(t + 1) * n] for t in range(tau)]
              + [vwords[t * n:(t + 1) * n] for t in range(tau)])
    kv_ref[0] = jnp.concatenate(pieces, axis=-1)


def _topk_body(windc_ref, windr_ref, fidx_ref, *, n, tau, k):
    b = pl.program_id(0)
    wcx = windc_ref[0, :, 0:1]
    wcy = windc_ref[0, :, 1:2]
    wrx = windr_ref[0, 0:1, :]
    wry = windr_ref[0, 1:2, :]
    dx = wcx - wrx
    dy = wcy - wry
    d2 = dx * dx + dy * dy
    rows = d2.shape[0]
    iota = lax.broadcasted_iota(jnp.int32, (rows, n), 1)
    cols = []
    for _ in range(k):
        j = jnp.argmin(d2, axis=1, keepdims=True).astype(jnp.int32)
        d2 = jnp.where(iota == j, jnp.inf, d2)
        cols.append(j + b * n)
    fidx_ref[0] = jnp.concatenate(cols, axis=1)


def _sc_gather_body(kv_hbm, idx_hbm, out_hbm, idx_v, rows_v, sem, *,
                    rows_per_worker, chunk, num_cores):
    wid = lax.axis_index("s") * num_cores + lax.axis_index("c")
    nchunks = rows_per_worker // chunk

    def body(c, carry):
        base = wid * rows_per_worker + c * chunk
        pltpu.sync_copy(idx_hbm.at[pl.ds(base, chunk)], idx_v)
        pltpu.async_copy(kv_hbm.at[idx_v], rows_v, sem).wait()
        pltpu.sync_copy(rows_v, out_hbm.at[pl.ds(base, chunk)])
        return carry

    lax.fori_loop(0, nchunks, body, 0)


def _attn_body(qcat_ref, g_ref, w1_ref, b1_ref, w2_ref, b2_ref,
               out_ref, *, d, tau, k):
    dw = d // 2
    td = tau * dw
    qcat = qcat_ref[0]
    g = g_ref[...]
    rows = qcat.shape[0]
    rk = rows * k
    hi_mask = jnp.int32(-65536)

    def lo(w):
        return lax.bitcast_convert_type(lax.shift_left(w, 16), jnp.float32)

    def hi(w):
        return lax.bitcast_convert_type(w & hi_mask, jnp.float32)

    def rep_rows(x):
        return jnp.broadcast_to(x[:, None, :],
                                (rows, k, x.shape[-1])).reshape(rk, x.shape[-1])

    lane = lax.broadcasted_iota(jnp.int32, (td, tau), 0)
    tcol = lax.broadcasted_iota(jnp.int32, (td, tau), 1)
    seg = (lane // dw == tcol).astype(jnp.float32)
    lane2 = lax.broadcasted_iota(jnp.int32, (tau, td), 1)
    trow = lax.broadcasted_iota(jnp.int32, (tau, td), 0)
    expand = (lane2 // dw == trow).astype(jnp.float32)
    lane3 = lax.broadcasted_iota(jnp.int32, (td, dw), 0)
    fcol = lax.broadcasted_iota(jnp.int32, (td, dw), 1)
    fold = (lane3 % dw == fcol).astype(jnp.float32)

    qet = rep_rows(qcat[:, :td])
    qot = rep_rows(qcat[:, td:])
    kw = g[:, :td]
    vw = g[:, td:]
    comb = qet * lo(kw) + qot * hi(kw)
    s4 = jnp.dot(comb, seg,
                 preferred_element_type=jnp.float32) / math.sqrt(d)
    s4g = s4.reshape(rows, k, tau)
    m = jnp.max(jnp.max(s4g, axis=2, keepdims=True), axis=1, keepdims=True)
    e = jnp.exp(s4g - m)
    ssum = jnp.sum(jnp.sum(e, axis=2, keepdims=True), axis=1, keepdims=True)
    w4 = (e / ssum).reshape(rk, tau)
    w128 = jnp.dot(w4, expand, preferred_element_type=jnp.float32)
    acc_e = jnp.dot(w128 * lo(vw), fold,
                    preferred_element_type=jnp.float32)
    acc_o = jnp.dot(w128 * hi(vw), fold,
                    preferred_element_type=jnp.float32)
    ctx_e = jnp.sum(acc_e.reshape(rows, k, dw), axis=1)
    ctx_o = jnp.sum(acc_o.reshape(rows, k, dw), axis=1)
    ctx = jnp.concatenate([ctx_e, ctx_o], axis=-1)
    hid = jnp.dot(ctx, w1_ref[...],
                  preferred_element_type=jnp.float32) + b1_ref[...]
    hid = 0.5 * hid * (1.0 + lax.erf(hid / math.sqrt(2.0)))
    out_ref[...] = jnp.dot(hid, w2_ref[...],
                           preferred_element_type=jnp.float32) + b2_ref[...]


def kernel(h_e, x_orig, Wq, bq, Wk, bk, Wv, bv, W1, b1, W2, b2):
    b, T, n, d = h_e.shape
    t0 = T - 1
    t_start = max(0, t0 - _TAU + 1)
    tau = t0 - t_start + 1
    k = min(_K, n)
    rows = _ROWS
    nb = n // rows
    kt = k * tau

    wind = x_orig[t0, :, :, 4:6]
    windr = jnp.transpose(wind, (0, 2, 1))
    h_last = h_e[:, t0]
    hist4 = h_e[:, t_start:t0 + 1]

    full = lambda shp: pl.BlockSpec(shp, lambda *_: (0,) * len(shp))

    na = 4
    nblk = n // na
    qcat, kv_packed = pl.pallas_call(
        functools.partial(_qkv_body, tau=tau, d=d),
        grid=(b, na),
        in_specs=[
            pl.BlockSpec((1, nblk, d), lambda i, r: (i, r, 0)),
            pl.BlockSpec((1, tau, nblk, d), lambda i, r: (i, 0, r, 0)),
            full((d, d)), full((1, d)), full((d, d)), full((1, d)),
            full((d, d)), full((1, d)),
        ],
        out_specs=[
            pl.BlockSpec((1, nblk, tau * d), lambda i, r: (i, r, 0)),
            pl.BlockSpec((1, nblk, tau * d), lambda i, r: (i, r, 0)),
        ],
        out_shape=[
            jax.ShapeDtypeStruct((b, n, tau * d), jnp.float32),
            jax.ShapeDtypeStruct((b, n, tau * d), jnp.int32),
        ],
    )(h_last, hist4, Wq, bq.reshape(1, d),
      Wk, bk.reshape(1, d), Wv, bv.reshape(1, d))
    kv_packed = kv_packed.reshape(b * n, tau * d)

    fidx = pl.pallas_call(
        functools.partial(_topk_body, n=n, tau=tau, k=k),
        grid=(b, nb),
        in_specs=[
            pl.BlockSpec((1, rows, 2), lambda i, r: (i, r, 0)),
            pl.BlockSpec((1, 2, n), lambda i, r: (i, 0, 0)),
        ],
        out_specs=pl.BlockSpec((1, rows, k), lambda i, r: (i, r, 0)),
        out_shape=jax.ShapeDtypeStruct((b, n, k), jnp.int32),
    )(wind, windr)

    try:
        info = plsc.get_sparse_core_info()
        num_cores, num_subcores = info.num_cores, info.num_subcores
    except ValueError:
        num_cores, num_subcores = 2, 16
    nw = num_cores * num_subcores
    batch_idx = n * k
    rpw = batch_idx // nw
    chunk = 128
    mesh = plsc.VectorSubcoreMesh(core_axis_name="c", subcore_axis_name="s",
                                  num_cores=num_cores,
                                  num_subcores=num_subcores)
    sc_gather = pl.kernel(
        functools.partial(_sc_gather_body, rows_per_worker=rpw, chunk=chunk,
                          num_cores=num_cores),
        out_type=jax.ShapeDtypeStruct((batch_idx, tau * d), jnp.int32),
        mesh=mesh,
        scratch_types=[
            pltpu.VMEM((chunk,), jnp.int32),
            pltpu.VMEM((chunk, tau * d), jnp.int32),
            pltpu.SemaphoreType.DMA,
        ],
    )

    w1p = jnp.concatenate([W1[0::2], W1[1::2]], axis=0)

    def attn_call(bi, g_b):
        return pl.pallas_call(
            functools.partial(_attn_body, d=d, tau=tau, k=k),
            grid=(nb,),
            in_specs=[
                pl.BlockSpec((1, rows, tau * d), lambda r: (bi, r, 0)),
                pl.BlockSpec((rows * k, tau * d), lambda r: (r, 0)),
                full((d, d)), full((d,)), full((d, d)), full((d,)),
            ],
            out_specs=pl.BlockSpec((rows, d), lambda r: (r, 0)),
            out_shape=jax.ShapeDtypeStruct((n, d), jnp.float32),
        )(qcat, g_b, w1p, b1, W2, b2)

    fidx_flat = fidx.reshape(b, batch_idx)
    outs = []
    for bi in range(b):
        g_b = sc_gather(kv_packed, fidx_flat[bi])
        outs.append(attn_call(bi, g_b))
    return jnp.stack(outs, axis=0)
